# baseline (device time: 57698 ns/iter reference)
import jax
import jax.numpy as jnp
from jax import lax
from jax.experimental import pallas as pl
from jax.experimental.pallas import tpu as pltpu

N_DEV = 16
N_FULL_HOPS = 7
N_SEG = 4

RING = (0, 4, 8, 12, 13, 9, 5, 1, 2, 6, 10, 14, 15, 11, 7, 3)


def kernel(x):
    m_per, n = x.shape
    seg = m_per // N_SEG

    ring = jnp.asarray(RING, jnp.int32)
    my = lax.axis_index("i")
    pos = jnp.argmax(ring == my).astype(jnp.int32)
    idx = jnp.arange(N_DEV, dtype=jnp.int32)
    fwd_origin = ring[(pos - idx) % N_DEV]
    bwd_origin = ring[(pos + idx) % N_DEV]
    params = jnp.concatenate(
        [
            ring[(pos + 1) % N_DEV][None],
            ring[(pos - 1) % N_DEV][None],
            fwd_origin[: N_FULL_HOPS + 2],
            bwd_origin[: N_FULL_HOPS + 2],
        ]
    )

    def body(params_ref, x_ref, out_ref, fs_sems, fr_sems, bs_sems, br_sems):
        right = params_ref[0]
        left = params_ref[1]

        def f_origin(h):
            return params_ref[2 + h]

        def b_origin(h):
            return params_ref[2 + (N_FULL_HOPS + 2) + h]

        barrier_sem = pltpu.get_barrier_semaphore()
        for nbr in (left, right):
            pl.semaphore_signal(
                barrier_sem, inc=1,
                device_id=(nbr,), device_id_type=pl.DeviceIdType.MESH,
            )
        pl.semaphore_wait(barrier_sem, 2)

        def seg_copy(origin, s, sems_pair, h, dev):
            sl = pl.ds(origin * m_per + s * seg, seg)
            return pltpu.make_async_remote_copy(
                src_ref=out_ref.at[sl],
                dst_ref=out_ref.at[sl],
                send_sem=sems_pair[0].at[h, s],
                recv_sem=sems_pair[1].at[h, s],
                device_id=(dev,),
                device_id_type=pl.DeviceIdType.MESH,
            )

        fwd = (fs_sems, fr_sems)
        bwd = (bs_sems, br_sems)

        def fwd_send(h, s):
            return seg_copy(f_origin(h), s, fwd, h, right)

        def fwd_recv(h, s):
            return seg_copy(f_origin(h + 1), s, fwd, h, left)

        def bwd_send(h, s):
            return seg_copy(b_origin(h), s, bwd, h, left)

        def bwd_recv(h, s):
            return seg_copy(b_origin(h + 1), s, bwd, h, right)

        sends = []

        def start(d):
            d.start()
            sends.append(d)

        my_row = params_ref[2]
        for k in range(N_SEG):
            for s in (k, N_SEG - 1 - k) if k < N_SEG // 2 else ():
                out_ref[pl.ds(my_row * m_per + s * seg, seg), :] = (
                    x_ref[pl.ds(s * seg, seg), :].astype(out_ref.dtype)
                )
            start(fwd_send(0, k))
            start(bwd_send(0, N_SEG - 1 - k))

        for h in range(1, N_FULL_HOPS):
            for k in range(N_SEG):
                fwd_recv(h - 1, k).wait_recv()
                start(fwd_send(h, k))
                bwd_recv(h - 1, N_SEG - 1 - k).wait_recv()
                start(bwd_send(h, N_SEG - 1 - k))

        h6 = N_FULL_HOPS - 1
        h7 = N_FULL_HOPS
        fwd_recv(h6, 0).wait_recv()
        start(fwd_send(h7, 0))
        bwd_recv(h6, 3).wait_recv()
        start(bwd_send(h7, 3))
        fwd_recv(h6, 1).wait_recv()
        start(fwd_send(h7, 1))
        bwd_recv(h6, 2).wait_recv()
        start(bwd_send(h7, 2))

        for s in (2, 3):
            fwd_recv(h6, s).wait_recv()
        for s in (1, 0):
            bwd_recv(h6, s).wait_recv()
        fwd_recv(h7, 0).wait_recv()
        fwd_recv(h7, 1).wait_recv()
        bwd_recv(h7, 3).wait_recv()
        bwd_recv(h7, 2).wait_recv()

        for d in sends:
            d.wait_send()

    return pl.pallas_call(
        body,
        out_shape=jax.ShapeDtypeStruct((N_DEV * m_per, n), jnp.bfloat16),
        in_specs=[
            pl.BlockSpec(memory_space=pltpu.SMEM),
            pl.BlockSpec(memory_space=pltpu.VMEM),
        ],
        out_specs=pl.BlockSpec(memory_space=pltpu.VMEM),
        scratch_shapes=[
            pltpu.SemaphoreType.DMA((N_FULL_HOPS + 1, N_SEG)),
            pltpu.SemaphoreType.DMA((N_FULL_HOPS + 1, N_SEG)),
            pltpu.SemaphoreType.DMA((N_FULL_HOPS + 1, N_SEG)),
            pltpu.SemaphoreType.DMA((N_FULL_HOPS + 1, N_SEG)),
        ],
        compiler_params=pltpu.CompilerParams(collective_id=0),
    )(params, x)


# device time: 51234 ns/iter; 1.1262x vs baseline; 1.1262x over previous
import jax
import jax.numpy as jnp
from jax import lax
from jax.experimental import pallas as pl
from jax.experimental.pallas import tpu as pltpu

N_DEV = 16
N_FULL_HOPS = 7
N_SEG = 4

RING = (0, 4, 8, 12, 13, 9, 5, 1, 2, 6, 10, 14, 15, 11, 7, 3)


def kernel(x):
    m_per, n = x.shape
    seg = m_per // N_SEG

    def body(x_ref, out_ref, fs_sems, fr_sems, bs_sems, br_sems):
        my = lax.axis_index("i")
        q = lax.rem(my, 4)
        z = lax.div(my, 4)
        q_even = lax.rem(q, 2) == 0
        pos = 4 * q + lax.select(q_even, z, 3 - z)

        def ring_at(p):
            p = lax.rem(p + 2 * N_DEV, N_DEV)
            pq = lax.div(p, 4)
            pr = lax.rem(p, 4)
            return pq + 4 * lax.select(lax.rem(pq, 2) == 0, pr, 3 - pr)

        right = ring_at(pos + 1)
        left = ring_at(pos - 1)

        def f_origin(h):
            return ring_at(pos - h)

        def b_origin(h):
            return ring_at(pos + h)

        barrier_sem = pltpu.get_barrier_semaphore()
        for nbr in (left, right):
            pl.semaphore_signal(
                barrier_sem, inc=1,
                device_id=(nbr,), device_id_type=pl.DeviceIdType.MESH,
            )
        pl.semaphore_wait(barrier_sem, 2)

        def seg_copy(origin, s, sems_pair, h, dev):
            sl = pl.ds(origin * m_per + s * seg, seg)
            return pltpu.make_async_remote_copy(
                src_ref=out_ref.at[sl],
                dst_ref=out_ref.at[sl],
                send_sem=sems_pair[0].at[h, s],
                recv_sem=sems_pair[1].at[h, s],
                device_id=(dev,),
                device_id_type=pl.DeviceIdType.MESH,
            )

        fwd = (fs_sems, fr_sems)
        bwd = (bs_sems, br_sems)

        def fwd_send(h, s):
            return seg_copy(f_origin(h), s, fwd, h, right)

        def fwd_recv(h, s):
            return seg_copy(f_origin(h + 1), s, fwd, h, left)

        def bwd_send(h, s):
            return seg_copy(b_origin(h), s, bwd, h, left)

        def bwd_recv(h, s):
            return seg_copy(b_origin(h + 1), s, bwd, h, right)

        sends = []

        def start(d):
            d.start()
            sends.append(d)

        my_row = my
        for k in range(N_SEG):
            for s in (k, N_SEG - 1 - k) if k < N_SEG // 2 else ():
                out_ref[pl.ds(my_row * m_per + s * seg, seg), :] = (
                    x_ref[pl.ds(s * seg, seg), :].astype(out_ref.dtype)
                )
            start(fwd_send(0, k))
            start(bwd_send(0, N_SEG - 1 - k))

        for h in range(1, N_FULL_HOPS):
            for k in range(N_SEG):
                fwd_recv(h - 1, k).wait_recv()
                start(fwd_send(h, k))
                bwd_recv(h - 1, N_SEG - 1 - k).wait_recv()
                start(bwd_send(h, N_SEG - 1 - k))

        h6 = N_FULL_HOPS - 1
        h7 = N_FULL_HOPS
        fwd_recv(h6, 0).wait_recv()
        start(fwd_send(h7, 0))
        bwd_recv(h6, 3).wait_recv()
        start(bwd_send(h7, 3))
        fwd_recv(h6, 1).wait_recv()
        start(fwd_send(h7, 1))
        bwd_recv(h6, 2).wait_recv()
        start(bwd_send(h7, 2))

        for s in (2, 3):
            fwd_recv(h6, s).wait_recv()
        for s in (1, 0):
            bwd_recv(h6, s).wait_recv()
        fwd_recv(h7, 0).wait_recv()
        fwd_recv(h7, 1).wait_recv()
        bwd_recv(h7, 3).wait_recv()
        bwd_recv(h7, 2).wait_recv()

        for d in sends:
            d.wait_send()

    return pl.pallas_call(
        body,
        out_shape=jax.ShapeDtypeStruct((N_DEV * m_per, n), jnp.bfloat16),
        in_specs=[pl.BlockSpec(memory_space=pltpu.VMEM)],
        out_specs=pl.BlockSpec(memory_space=pltpu.VMEM),
        scratch_shapes=[
            pltpu.SemaphoreType.DMA((N_FULL_HOPS + 1, N_SEG)),
            pltpu.SemaphoreType.DMA((N_FULL_HOPS + 1, N_SEG)),
            pltpu.SemaphoreType.DMA((N_FULL_HOPS + 1, N_SEG)),
            pltpu.SemaphoreType.DMA((N_FULL_HOPS + 1, N_SEG)),
        ],
        compiler_params=pltpu.CompilerParams(collective_id=0),
    )(x)
